# baseline (speedup 1.0000x reference)
import functools

import jax
import jax.numpy as jnp
from jax import lax
from jax.experimental import pallas as pl
from jax.experimental.pallas import tpu as pltpu
from jax.experimental.pallas import tpu_sc as plsc

N = 10000
E = 320000
HID = 64
IN_F = 128
OUT_F = 128
NORM_FACTOR = 100.0
NORM_CONST = 1.0
COORDS_RANGE = 15.0

NBLK = 1000
EBLK = 2560


def _silu(v):
    return v * jax.nn.sigmoid(v)


def _full(shape):
    return pl.BlockSpec(shape, lambda i: tuple(0 for _ in shape))


def _emb_in_body(x_ref, we_ref, be_ref, wr_ref, wc_ref, h_ref, a_ref, b_ref):
    h = x_ref[...] @ we_ref[...] + be_ref[...]
    h_ref[...] = h
    a_ref[...] = h @ wr_ref[...]
    b_ref[...] = h @ wc_ref[...]


def _emb_in(x, We, be, W1r, W1c):
    return pl.pallas_call(
        _emb_in_body,
        grid=(N // NBLK,),
        in_specs=[
            pl.BlockSpec((NBLK, IN_F), lambda i: (i, 0)),
            _full((IN_F, HID)), _full((HID,)),
            _full((HID, HID)), _full((HID, HID)),
        ],
        out_specs=[
            pl.BlockSpec((NBLK, HID), lambda i: (i, 0)),
            pl.BlockSpec((NBLK, HID), lambda i: (i, 0)),
            pl.BlockSpec((NBLK, HID), lambda i: (i, 0)),
        ],
        out_shape=[
            jax.ShapeDtypeStruct((N, HID), jnp.float32),
            jax.ShapeDtypeStruct((N, HID), jnp.float32),
            jax.ShapeDtypeStruct((N, HID), jnp.float32),
        ],
    )(x, We, be, W1r, W1c)


def _node_update_body(h_ref, s0_ref, s1_ref, mask_ref, wn1h_ref, wn1a_ref,
                      bn1_ref, wn2_ref, bn2_ref, *proj_refs):
    nproj = (len(proj_refs) - 1) // 2
    h = h_ref[...]
    agg = (s0_ref[...] + s1_ref[...]) * (1.0 / NORM_FACTOR)
    u = _silu(h @ wn1h_ref[...] + agg @ wn1a_ref[...] + bn1_ref[...])
    hn = (h + u @ wn2_ref[...] + bn2_ref[...]) * mask_ref[...]
    out_refs = proj_refs[nproj:]
    out_refs[0][...] = hn
    for k in range(nproj):
        out_refs[1 + k][...] = hn @ proj_refs[k][...]


def _node_update(h, s2n, mask, Wn1h, Wn1a, bn1, Wn2, bn2, proj_ws):
    nproj = len(proj_ws)
    return pl.pallas_call(
        _node_update_body,
        grid=(N // NBLK,),
        in_specs=[
            pl.BlockSpec((NBLK, HID), lambda i: (i, 0)),
            pl.BlockSpec((NBLK, HID), lambda i: (i, 0)),
            pl.BlockSpec((NBLK, HID), lambda i: (i + N // NBLK, 0)),
            pl.BlockSpec((NBLK, 1), lambda i: (i, 0)),
            _full((HID, HID)), _full((HID, HID)), _full((HID,)),
            _full((HID, HID)), _full((HID,)),
        ] + [_full((HID, HID)) for _ in range(nproj)],
        out_specs=[pl.BlockSpec((NBLK, HID), lambda i: (i, 0))
                   for _ in range(1 + nproj)],
        out_shape=[jax.ShapeDtypeStruct((N, HID), jnp.float32)
                   for _ in range(1 + nproj)],
    )(h, s2n, s2n, mask, Wn1h, Wn1a, bn1, Wn2, bn2, *proj_ws)


def _emb_out_body(h_ref, w_ref, b_ref, mask_ref, o_ref):
    o_ref[...] = (h_ref[...] @ w_ref[...] + b_ref[...]) * mask_ref[...]


def _emb_out(h, Wo, bo, mask):
    return pl.pallas_call(
        _emb_out_body,
        grid=(N // NBLK,),
        in_specs=[
            pl.BlockSpec((NBLK, HID), lambda i: (i, 0)),
            _full((HID, OUT_F)), _full((OUT_F,)),
            pl.BlockSpec((NBLK, 1), lambda i: (i, 0)),
        ],
        out_specs=pl.BlockSpec((NBLK, OUT_F), lambda i: (i, 0)),
        out_shape=jax.ShapeDtypeStruct((N, OUT_F), jnp.float32),
    )(h, Wo, bo, mask)


E2 = E // 2
EB2 = 1280


def _edge_mlp_body(pre_ref, f_ref, w1_ref, b1_ref, w2_ref, b2_ref, o_ref):
    z = _silu(pre_ref[...] + f_ref[...] @ w1_ref[...] + b1_ref[...])
    o_ref[...] = _silu(z @ w2_ref[...] + b2_ref[...])


def _edge_mlp(pre_p, F, W1bd, b1p, W2bd, b2p):
    return pl.pallas_call(
        _edge_mlp_body,
        grid=(E2 // EB2,),
        in_specs=[
            pl.BlockSpec((EB2, 128), lambda i: (i, 0)),
            pl.BlockSpec((EB2, 128), lambda i: (i, 0)),
            _full((128, 128)), _full((128,)),
            _full((128, 128)), _full((128,)),
        ],
        out_specs=pl.BlockSpec((EB2, 128), lambda i: (i, 0)),
        out_shape=jax.ShapeDtypeStruct((E2, 128), jnp.float32),
    )(pre_p, F, W1bd, b1p, W2bd, b2p)


def _coord_mlp_body(pre_ref, f_ref, g_ref, w1_ref, b1_ref, w2_ref, b2_ref,
                    w3_ref, rsel_ref, o_ref):
    z = _silu(pre_ref[...] + f_ref[...] @ w1_ref[...] + b1_ref[...])
    z = _silu(z @ w2_ref[...] + b2_ref[...])
    phi = jnp.tanh(z @ w3_ref[...]) * COORDS_RANGE
    g = g_ref[...]
    radial = g @ rsel_ref[...]
    scale = phi / (jnp.sqrt(radial + 1e-8) + NORM_CONST)
    lane = lax.broadcasted_iota(jnp.int32, g.shape, 1) % 64
    keep = jnp.logical_and(lane >= 1, lane <= 3).astype(jnp.float32)
    o_ref[...] = g * scale * keep


def _coord_mlp(pre_p, F, G, W1bd, b1p, W2bd, b2p, W3sel, RSEL):
    return pl.pallas_call(
        _coord_mlp_body,
        grid=(E2 // EB2,),
        in_specs=[
            pl.BlockSpec((EB2, 128), lambda i: (i, 0)),
            pl.BlockSpec((EB2, 128), lambda i: (i, 0)),
            pl.BlockSpec((EB2, 128), lambda i: (i, 0)),
            _full((128, 128)), _full((128,)),
            _full((128, 128)), _full((128,)),
            _full((128, 128)), _full((128, 128)),
        ],
        out_specs=pl.BlockSpec((EB2, 128), lambda i: (i, 0)),
        out_shape=jax.ShapeDtypeStruct((E2, 128), jnp.float32),
    )(pre_p, F, G, W1bd, b1p, W2bd, b2p, W3sel, RSEL)


def _pad_feat_w(W17):
    Z = jnp.zeros((HID, HID), jnp.float32)
    return Z.at[0, :].set(W17[0]).at[4:20, :].set(W17[1:17])


def _blockdiag(W):
    Z = jnp.zeros((HID, HID), jnp.float32)
    return jnp.block([[W, Z], [Z, W]])


NW = 32
EPW = E // NW
CCH = 1000


def _sc_mesh():
    return plsc.VectorSubcoreMesh(core_axis_name="c", subcore_axis_name="s")


CCB = 400
NPAIR = (EPW // CCB) // 2


def _combine(A, B, row, col, pxyz=None, ea=None):
    with_geom = pxyz is not None
    ccb = 400
    nch = EPW // ccb
    npair = nch // 2
    has_tail = nch % 2 == 1
    out_type = [jax.ShapeDtypeStruct((E, HID), jnp.float32)]
    scratch = [
        pltpu.VMEM((ccb,), jnp.int32), pltpu.VMEM((ccb,), jnp.int32),
        pltpu.VMEM((ccb,), jnp.int32), pltpu.VMEM((ccb,), jnp.int32),
        pltpu.VMEM((ccb, HID), jnp.float32),
        pltpu.VMEM((ccb, HID), jnp.float32),
    ] + [pltpu.SemaphoreType.DMA] * 6
    if with_geom:
        out_type.append(jax.ShapeDtypeStruct((E * HID,), jnp.float32))
        scratch += [
            pltpu.VMEM((N,), jnp.float32), pltpu.VMEM((N,), jnp.float32),
            pltpu.VMEM((N,), jnp.float32),
            pltpu.VMEM((ccb * HID,), jnp.float32),
            pltpu.VMEM((ccb, 16), jnp.float32),
        ]

    @functools.partial(
        pl.kernel,
        out_type=tuple(out_type) if with_geom else out_type[0],
        mesh=_sc_mesh(),
        scratch_types=scratch,
        compiler_params=pltpu.CompilerParams(
            use_tc_tiling_on_sc=False, needs_layout_passes=False),
    )
    def k(*refs):
        if with_geom:
            (a_hbm, b_hbm, row_hbm, col_hbm, px_hbm, py_hbm, pz_hbm,
             ea_hbm, o_hbm, g_hbm,
             rid0, cid0, rid1, cid1, buf0, buf1,
             sa0, sa1, sb0, sb1, sw0, sw1,
             pxv, pyv, pzv, gbuf, eabuf) = refs
        else:
            (a_hbm, b_hbm, row_hbm, col_hbm, o_hbm,
             rid0, cid0, rid1, cid1, buf0, buf1,
             sa0, sa1, sb0, sb1, sw0, sw1) = refs
        wid = lax.axis_index("s") * 2 + lax.axis_index("c")
        base = wid * EPW
        if with_geom:
            pltpu.sync_copy(px_hbm, pxv)
            pltpu.sync_copy(py_hbm, pyv)
            pltpu.sync_copy(pz_hbm, pzv)
            zv = jnp.zeros((16,), jnp.float32)

            def zinit(i, carry):
                gbuf[pl.ds(i * 16, 16)] = zv
                return carry

            lax.fori_loop(0, ccb * HID // 16, zinit, 0)

        def geom_chunk(off, rid, cid, gbuf):
            pltpu.sync_copy(ea_hbm.at[pl.ds(off, ccb)], eabuf)

            def grp(g, carry2):
                rvec = rid[pl.ds(g * 16, 16)]
                cvec = cid[pl.ds(g * 16, 16)]
                dx = plsc.load_gather(pxv, [rvec]) - plsc.load_gather(pxv, [cvec])
                dy = plsc.load_gather(pyv, [rvec]) - plsc.load_gather(pyv, [cvec])
                dz = plsc.load_gather(pzv, [rvec]) - plsc.load_gather(pzv, [cvec])
                dist = dx * dx + dy * dy + dz * dz
                iot = lax.iota(jnp.int32, 16)
                rows = g * (16 * HID) + iot * HID
                plsc.store_scatter(gbuf, [rows], dist)
                plsc.store_scatter(gbuf, [rows + 1], dx)
                plsc.store_scatter(gbuf, [rows + 2], dy)
                plsc.store_scatter(gbuf, [rows + 3], dz)
                erow = g * 16 + iot
                for f in range(16):
                    fv = plsc.load_gather(eabuf, [erow, iot * 0 + f])
                    plsc.store_scatter(gbuf, [rows + 4 + f], fv)
                return carry2

            lax.fori_loop(0, ccb // 16, grp, 0)

        def do_chunk_sync(off, rid, cid, buf, gbuf, sa, sb, sw):
            pltpu.sync_copy(row_hbm.at[pl.ds(off, ccb)], rid)
            pltpu.sync_copy(col_hbm.at[pl.ds(off, ccb)], cid)
            pltpu.async_copy(a_hbm.at[rid], buf, sa).wait()
            cb = pltpu.async_copy(b_hbm.at[cid], buf, sb, add=True)
            if with_geom:
                geom_chunk(off, rid, cid, gbuf)
            cb.wait()
            w = pltpu.async_copy(buf, o_hbm.at[pl.ds(off, ccb)], sw)
            if with_geom:
                pltpu.sync_copy(gbuf, g_hbm.at[pl.ds(off * HID, ccb * HID)])
            w.wait()

        def pair(pp, carry):
            off0 = base + pp * 2 * ccb
            off1 = off0 + ccb
            pltpu.sync_copy(row_hbm.at[pl.ds(off0, ccb)], rid0)
            pltpu.sync_copy(col_hbm.at[pl.ds(off0, ccb)], cid0)
            pltpu.sync_copy(row_hbm.at[pl.ds(off1, ccb)], rid1)
            pltpu.sync_copy(col_hbm.at[pl.ds(off1, ccb)], cid1)
            a0 = pltpu.async_copy(a_hbm.at[rid0], buf0, sa0)
            a1 = pltpu.async_copy(a_hbm.at[rid1], buf1, sa1)
            a0.wait()
            b0 = pltpu.async_copy(b_hbm.at[cid0], buf0, sb0, add=True)
            a1.wait()
            b1 = pltpu.async_copy(b_hbm.at[cid1], buf1, sb1, add=True)
            if with_geom:
                geom_chunk(off0, rid0, cid0, gbuf)
            b0.wait()
            w0 = pltpu.async_copy(buf0, o_hbm.at[pl.ds(off0, ccb)], sw0)
            if with_geom:
                pltpu.sync_copy(gbuf, g_hbm.at[pl.ds(off0 * HID, ccb * HID)])
                geom_chunk(off1, rid1, cid1, gbuf)
            b1.wait()
            w1 = pltpu.async_copy(buf1, o_hbm.at[pl.ds(off1, ccb)], sw1)
            if with_geom:
                pltpu.sync_copy(gbuf, g_hbm.at[pl.ds(off1 * HID, ccb * HID)])
            w0.wait()
            w1.wait()
            return carry

        lax.fori_loop(0, npair, pair, 0)
        if has_tail:
            do_chunk_sync(base + 2 * npair * ccb, rid0, cid0, buf0,
                          gbuf if with_geom else None, sa0, sb0, sw0)

    if with_geom:
        return k(A, B, row, col, *pxyz, ea)
    return k(A, B, row, col)


def _segsum(vals, row, K, zeros):
    npr = N // 16

    @functools.partial(
        pl.kernel,
        out_type=jax.ShapeDtypeStruct((2 * N, K), jnp.float32),
        mesh=_sc_mesh(),
        scratch_types=[
            pltpu.VMEM((CCB,), jnp.int32),
            pltpu.VMEM((CCB,), jnp.int32),
            pltpu.VMEM((CCB, K), jnp.float32),
            pltpu.VMEM((CCB, K), jnp.float32),
            pltpu.VMEM_SHARED((N, K), jnp.float32),
        ] + [pltpu.SemaphoreType.DMA] * 4,
        compiler_params=pltpu.CompilerParams(use_tc_tiling_on_sc=False),
    )
    def k(vals_hbm, row_hbm, zeros_hbm, out_hbm,
          rid0, rid1, vbuf0, vbuf1, acc, sv0, sv1, ss0, ss1):
        ci = lax.axis_index("c")
        sid = lax.axis_index("s")
        wid = sid * 2 + ci
        pltpu.sync_copy(zeros_hbm.at[pl.ds(sid * npr, npr)],
                        acc.at[pl.ds(sid * npr, npr)])
        plsc.subcore_barrier()
        base = wid * EPW

        def pair(p, carry):
            off0 = base + p * 2 * CCB
            off1 = off0 + CCB
            pltpu.sync_copy(row_hbm.at[pl.ds(off0, CCB)], rid0)
            pltpu.sync_copy(row_hbm.at[pl.ds(off1, CCB)], rid1)
            v0 = pltpu.async_copy(vals_hbm.at[pl.ds(off0, CCB)], vbuf0, sv0)
            v1 = pltpu.async_copy(vals_hbm.at[pl.ds(off1, CCB)], vbuf1, sv1)
            v0.wait()
            c0 = pltpu.async_copy(vbuf0, acc.at[rid0], ss0, add=True)
            v1.wait()
            c1 = pltpu.async_copy(vbuf1, acc.at[rid1], ss1, add=True)
            c0.wait()
            c1.wait()
            return carry

        lax.fori_loop(0, NPAIR, pair, 0)
        off = base + 2 * NPAIR * CCB
        pltpu.sync_copy(row_hbm.at[pl.ds(off, CCB)], rid0)
        pltpu.sync_copy(vals_hbm.at[pl.ds(off, CCB)], vbuf0)
        pltpu.sync_copy(vbuf0, acc.at[rid0], add=True)
        plsc.subcore_barrier()
        pltpu.sync_copy(acc.at[pl.ds(sid * npr, npr)],
                        out_hbm.at[pl.ds(ci * N + sid * npr, npr)])

    return k(vals, row, zeros)


def _split_w1(W1):
    return W1[:HID], W1[HID:2 * HID], W1[2 * HID:]


def kernel(x, pos, mask, edge_attr, params, edge_index):
    row = edge_index[0]
    col = edge_index[1]
    blocks = params['blocks']

    gcl0 = blocks[0]['gcls'][0]
    W1_0 = gcl0['edge_mlp'][0][0]
    W1r0, W1c0, _ = _split_w1(W1_0)
    We, be = params['emb_in']
    h, A, B = _emb_in(x, We, be, W1r0, W1c0)
    zeros64 = jnp.zeros((N, HID), jnp.float32)
    ones_row = jnp.ones((1, HID), jnp.float32)
    SEL0 = jnp.zeros((HID, HID), jnp.float32).at[0, :].set(1.0)
    RSEL = _blockdiag(SEL0)
    F = None

    for bi, blk in enumerate(blocks):
        gcl = blk['gcls'][0]
        (W1, b1), (W2, b2) = gcl['edge_mlp']
        _, _, W1e = _split_w1(W1)
        pre, gflat = _combine(A, B, row, col,
                              (pos[:, 0], pos[:, 1], pos[:, 2]), edge_attr)
        G = gflat.reshape(E2, 128)
        if F is None:
            F = G
        W1e_pad = _pad_feat_w(W1e)
        m_p = _edge_mlp(pre.reshape(E2, 128), F,
                        _blockdiag(W1e_pad), jnp.concatenate([b1, b1]),
                        _blockdiag(W2), jnp.concatenate([b2, b2]))
        s2n = _segsum(m_p.reshape(E, HID), row, HID, zeros64)

        (Wn1, bn1), (Wn2, bn2) = gcl['node_mlp']
        Wn1h, Wn1a = Wn1[:HID], Wn1[HID:]
        (Wc1, bc1), (Wc2, bc2), (Wc3,) = blk['coord_mlp']
        Wc1r, Wc1c, Wc1e = _split_w1(Wc1)
        proj_ws = [Wc1r, Wc1c]
        if bi + 1 < len(blocks):
            gcl_n = blocks[bi + 1]['gcls'][0]
            W1n = gcl_n['edge_mlp'][0][0]
            W1rn, W1cn, _ = _split_w1(W1n)
            proj_ws += [W1rn, W1cn]
        outs = _node_update(h, s2n, mask, Wn1h, Wn1a, bn1, Wn2, bn2, proj_ws)
        h, Ac, Bc = outs[0], outs[1], outs[2]
        if bi + 1 < len(blocks):
            A, B = outs[3], outs[4]

        pre_c = _combine(Ac, Bc, row, col)
        Wc1e_pad = _pad_feat_w(Wc1e)
        W3sel = _blockdiag(Wc3 @ ones_row)
        trans_p = _coord_mlp(pre_c.reshape(E2, 128), F, G,
                             _blockdiag(Wc1e_pad), jnp.concatenate([bc1, bc1]),
                             _blockdiag(Wc2), jnp.concatenate([bc2, bc2]),
                             W3sel, RSEL)
        c2n = _segsum(trans_p.reshape(E, HID), row, HID, zeros64)
        cagg = c2n[:N, 1:4] + c2n[N:, 1:4]
        pos = pos + (cagg * (1.0 / NORM_FACTOR)) * mask
        h = h * mask

    Wo, bo = params['emb_out']
    return (_emb_out(h, Wo, bo, mask), pos)


# speedup vs baseline: 1.1204x; 1.1204x over previous
import functools

import jax
import jax.numpy as jnp
from jax import lax
from jax.experimental import pallas as pl
from jax.experimental.pallas import tpu as pltpu
from jax.experimental.pallas import tpu_sc as plsc

N = 10000
E = 320000
HID = 64
IN_F = 128
OUT_F = 128
NORM_FACTOR = 100.0
NORM_CONST = 1.0
COORDS_RANGE = 15.0

NBLK = 1000
EBLK = 2560


def _silu(v):
    return v * jax.nn.sigmoid(v)


def _full(shape):
    return pl.BlockSpec(shape, lambda i: tuple(0 for _ in shape))


def _emb_in_body(x_ref, we_ref, be_ref, wr_ref, wc_ref, h_ref, a_ref, b_ref):
    h = x_ref[...] @ we_ref[...] + be_ref[...]
    h_ref[...] = h
    a_ref[...] = h @ wr_ref[...]
    b_ref[...] = h @ wc_ref[...]


def _emb_in(x, We, be, W1r, W1c):
    return pl.pallas_call(
        _emb_in_body,
        grid=(N // NBLK,),
        in_specs=[
            pl.BlockSpec((NBLK, IN_F), lambda i: (i, 0)),
            _full((IN_F, HID)), _full((HID,)),
            _full((HID, HID)), _full((HID, HID)),
        ],
        out_specs=[
            pl.BlockSpec((NBLK, HID), lambda i: (i, 0)),
            pl.BlockSpec((NBLK, HID), lambda i: (i, 0)),
            pl.BlockSpec((NBLK, HID), lambda i: (i, 0)),
        ],
        out_shape=[
            jax.ShapeDtypeStruct((N, HID), jnp.float32),
            jax.ShapeDtypeStruct((N, HID), jnp.float32),
            jax.ShapeDtypeStruct((N, HID), jnp.float32),
        ],
    )(x, We, be, W1r, W1c)


def _node_update_body(h_ref, s0_ref, s1_ref, mask_ref, wn1h_ref, wn1a_ref,
                      bn1_ref, wn2_ref, bn2_ref, *proj_refs):
    nproj = (len(proj_refs) - 1) // 2
    h = h_ref[...]
    agg = (s0_ref[...] + s1_ref[...]) * (1.0 / NORM_FACTOR)
    u = _silu(h @ wn1h_ref[...] + agg @ wn1a_ref[...] + bn1_ref[...])
    hn = (h + u @ wn2_ref[...] + bn2_ref[...]) * mask_ref[...]
    out_refs = proj_refs[nproj:]
    out_refs[0][...] = hn
    for k in range(nproj):
        out_refs[1 + k][...] = hn @ proj_refs[k][...]


def _node_update(h, s2n, mask, Wn1h, Wn1a, bn1, Wn2, bn2, proj_ws):
    nproj = len(proj_ws)
    return pl.pallas_call(
        _node_update_body,
        grid=(N // NBLK,),
        in_specs=[
            pl.BlockSpec((NBLK, HID), lambda i: (i, 0)),
            pl.BlockSpec((NBLK, HID), lambda i: (i, 0)),
            pl.BlockSpec((NBLK, HID), lambda i: (i + N // NBLK, 0)),
            pl.BlockSpec((NBLK, 1), lambda i: (i, 0)),
            _full((HID, HID)), _full((HID, HID)), _full((HID,)),
            _full((HID, HID)), _full((HID,)),
        ] + [_full((HID, HID)) for _ in range(nproj)],
        out_specs=[pl.BlockSpec((NBLK, HID), lambda i: (i, 0))
                   for _ in range(1 + nproj)],
        out_shape=[jax.ShapeDtypeStruct((N, HID), jnp.float32)
                   for _ in range(1 + nproj)],
    )(h, s2n, s2n, mask, Wn1h, Wn1a, bn1, Wn2, bn2, *proj_ws)


def _emb_out_body(h_ref, w_ref, b_ref, mask_ref, o_ref):
    o_ref[...] = (h_ref[...] @ w_ref[...] + b_ref[...]) * mask_ref[...]


def _emb_out(h, Wo, bo, mask):
    return pl.pallas_call(
        _emb_out_body,
        grid=(N // NBLK,),
        in_specs=[
            pl.BlockSpec((NBLK, HID), lambda i: (i, 0)),
            _full((HID, OUT_F)), _full((OUT_F,)),
            pl.BlockSpec((NBLK, 1), lambda i: (i, 0)),
        ],
        out_specs=pl.BlockSpec((NBLK, OUT_F), lambda i: (i, 0)),
        out_shape=jax.ShapeDtypeStruct((N, OUT_F), jnp.float32),
    )(h, Wo, bo, mask)


E2 = E // 2
EB2 = 1280


def _edge_mlp_body(pre_ref, f_ref, w1_ref, b1_ref, w2_ref, b2_ref, o_ref):
    z = _silu(pre_ref[...] + f_ref[...] @ w1_ref[...] + b1_ref[...])
    o_ref[...] = _silu(z @ w2_ref[...] + b2_ref[...])


def _edge_mlp(pre_p, F, W1bd, b1p, W2bd, b2p):
    return pl.pallas_call(
        _edge_mlp_body,
        grid=(E2 // EB2,),
        in_specs=[
            pl.BlockSpec((EB2, 128), lambda i: (i, 0)),
            pl.BlockSpec((EB2, 128), lambda i: (i, 0)),
            _full((128, 128)), _full((128,)),
            _full((128, 128)), _full((128,)),
        ],
        out_specs=pl.BlockSpec((EB2, 128), lambda i: (i, 0)),
        out_shape=jax.ShapeDtypeStruct((E2, 128), jnp.float32),
    )(pre_p, F, W1bd, b1p, W2bd, b2p)


def _coord_mlp_body(pre_ref, f_ref, g_ref, w1_ref, b1_ref, w2_ref, b2_ref,
                    w3_ref, rsel_ref, o_ref):
    z = _silu(pre_ref[...] + f_ref[...] @ w1_ref[...] + b1_ref[...])
    z = _silu(z @ w2_ref[...] + b2_ref[...])
    phi = jnp.tanh(z @ w3_ref[...]) * COORDS_RANGE
    g = g_ref[...]
    radial = g @ rsel_ref[...]
    scale = phi / (jnp.sqrt(radial + 1e-8) + NORM_CONST)
    lane = lax.broadcasted_iota(jnp.int32, g.shape, 1) % 64
    keep = jnp.logical_and(lane >= 1, lane <= 3).astype(jnp.float32)
    o_ref[...] = g * scale * keep


def _coord_mlp(pre_p, F, G, W1bd, b1p, W2bd, b2p, W3sel, RSEL):
    return pl.pallas_call(
        _coord_mlp_body,
        grid=(E2 // EB2,),
        in_specs=[
            pl.BlockSpec((EB2, 128), lambda i: (i, 0)),
            pl.BlockSpec((EB2, 128), lambda i: (i, 0)),
            pl.BlockSpec((EB2, 128), lambda i: (i, 0)),
            _full((128, 128)), _full((128,)),
            _full((128, 128)), _full((128,)),
            _full((128, 128)), _full((128, 128)),
        ],
        out_specs=pl.BlockSpec((EB2, 128), lambda i: (i, 0)),
        out_shape=jax.ShapeDtypeStruct((E2, 128), jnp.float32),
    )(pre_p, F, G, W1bd, b1p, W2bd, b2p, W3sel, RSEL)


def _pad_feat_w(W17):
    Z = jnp.zeros((HID, HID), jnp.float32)
    return Z.at[0, :].set(W17[0]).at[4:20, :].set(W17[1:17])


def _blockdiag(W):
    Z = jnp.zeros((HID, HID), jnp.float32)
    return jnp.block([[W, Z], [Z, W]])


NW = 32
EPW = E // NW
CCH = 1000


def _sc_mesh():
    return plsc.VectorSubcoreMesh(core_axis_name="c", subcore_axis_name="s")


CCB = 400
NPAIR = (EPW // CCB) // 2


def _combine(A, B, row, col, pxyz=None, ea=None):
    with_geom = pxyz is not None
    with_ea = ea is not None
    ccb = 400
    nch = EPW // ccb
    npair = nch // 2
    has_tail = nch % 2 == 1
    out_type = [jax.ShapeDtypeStruct((E, HID), jnp.float32)]
    scratch = [
        pltpu.VMEM((ccb,), jnp.int32), pltpu.VMEM((ccb,), jnp.int32),
        pltpu.VMEM((ccb,), jnp.int32), pltpu.VMEM((ccb,), jnp.int32),
        pltpu.VMEM((ccb, HID), jnp.float32),
        pltpu.VMEM((ccb, HID), jnp.float32),
    ] + [pltpu.SemaphoreType.DMA] * 6
    if with_geom:
        out_type.append(jax.ShapeDtypeStruct((E * HID,), jnp.float32))
        scratch += [
            pltpu.VMEM((N,), jnp.float32), pltpu.VMEM((N,), jnp.float32),
            pltpu.VMEM((N,), jnp.float32),
            pltpu.VMEM((ccb * HID,), jnp.float32),
        ]
        if with_ea:
            scratch += [pltpu.VMEM((ccb, 16), jnp.float32)]

    @functools.partial(
        pl.kernel,
        out_type=tuple(out_type) if with_geom else out_type[0],
        mesh=_sc_mesh(),
        scratch_types=scratch,
        compiler_params=pltpu.CompilerParams(
            use_tc_tiling_on_sc=False, needs_layout_passes=False),
    )
    def k(*refs):
        if with_geom:
            if with_ea:
                (a_hbm, b_hbm, row_hbm, col_hbm, px_hbm, py_hbm, pz_hbm,
                 ea_hbm, o_hbm, g_hbm,
                 rid0, cid0, rid1, cid1, buf0, buf1,
                 sa0, sa1, sb0, sb1, sw0, sw1,
                 pxv, pyv, pzv, gbuf, eabuf) = refs
            else:
                (a_hbm, b_hbm, row_hbm, col_hbm, px_hbm, py_hbm, pz_hbm,
                 o_hbm, g_hbm,
                 rid0, cid0, rid1, cid1, buf0, buf1,
                 sa0, sa1, sb0, sb1, sw0, sw1,
                 pxv, pyv, pzv, gbuf) = refs
        else:
            (a_hbm, b_hbm, row_hbm, col_hbm, o_hbm,
             rid0, cid0, rid1, cid1, buf0, buf1,
             sa0, sa1, sb0, sb1, sw0, sw1) = refs
        wid = lax.axis_index("s") * 2 + lax.axis_index("c")
        base = wid * EPW
        if with_geom:
            pltpu.sync_copy(px_hbm, pxv)
            pltpu.sync_copy(py_hbm, pyv)
            pltpu.sync_copy(pz_hbm, pzv)
            zv = jnp.zeros((16,), jnp.float32)

            def zinit(i, carry):
                gbuf[pl.ds(i * 16, 16)] = zv
                return carry

            lax.fori_loop(0, ccb * HID // 16, zinit, 0)

        def geom_chunk(off, rid, cid, gbuf):
            if with_ea:
                pltpu.sync_copy(ea_hbm.at[pl.ds(off, ccb)], eabuf)

            def grp(g, carry2):
                rvec = rid[pl.ds(g * 16, 16)]
                cvec = cid[pl.ds(g * 16, 16)]
                dx = plsc.load_gather(pxv, [rvec]) - plsc.load_gather(pxv, [cvec])
                dy = plsc.load_gather(pyv, [rvec]) - plsc.load_gather(pyv, [cvec])
                dz = plsc.load_gather(pzv, [rvec]) - plsc.load_gather(pzv, [cvec])
                dist = dx * dx + dy * dy + dz * dz
                iot = lax.iota(jnp.int32, 16)
                rows = g * (16 * HID) + iot * HID
                plsc.store_scatter(gbuf, [rows], dist)
                plsc.store_scatter(gbuf, [rows + 1], dx)
                plsc.store_scatter(gbuf, [rows + 2], dy)
                plsc.store_scatter(gbuf, [rows + 3], dz)
                return carry2

            lax.fori_loop(0, ccb // 16, grp, 0)
            if with_ea:
                def emerge(e, carry3):
                    gbuf[pl.ds(e * HID + 4, 16)] = eabuf[e]
                    return carry3

                lax.fori_loop(0, ccb, emerge, 0)

        def do_chunk_sync(off, rid, cid, buf, gbuf, sa, sb, sw):
            pltpu.sync_copy(row_hbm.at[pl.ds(off, ccb)], rid)
            pltpu.sync_copy(col_hbm.at[pl.ds(off, ccb)], cid)
            pltpu.async_copy(a_hbm.at[rid], buf, sa).wait()
            cb = pltpu.async_copy(b_hbm.at[cid], buf, sb, add=True)
            if with_geom:
                geom_chunk(off, rid, cid, gbuf)
            cb.wait()
            w = pltpu.async_copy(buf, o_hbm.at[pl.ds(off, ccb)], sw)
            if with_geom:
                pltpu.sync_copy(gbuf, g_hbm.at[pl.ds(off * HID, ccb * HID)])
            w.wait()

        def pair(pp, carry):
            off0 = base + pp * 2 * ccb
            off1 = off0 + ccb
            pltpu.sync_copy(row_hbm.at[pl.ds(off0, ccb)], rid0)
            pltpu.sync_copy(col_hbm.at[pl.ds(off0, ccb)], cid0)
            pltpu.sync_copy(row_hbm.at[pl.ds(off1, ccb)], rid1)
            pltpu.sync_copy(col_hbm.at[pl.ds(off1, ccb)], cid1)
            a0 = pltpu.async_copy(a_hbm.at[rid0], buf0, sa0)
            a1 = pltpu.async_copy(a_hbm.at[rid1], buf1, sa1)
            a0.wait()
            b0 = pltpu.async_copy(b_hbm.at[cid0], buf0, sb0, add=True)
            a1.wait()
            b1 = pltpu.async_copy(b_hbm.at[cid1], buf1, sb1, add=True)
            if with_geom:
                geom_chunk(off0, rid0, cid0, gbuf)
            b0.wait()
            w0 = pltpu.async_copy(buf0, o_hbm.at[pl.ds(off0, ccb)], sw0)
            if with_geom:
                pltpu.sync_copy(gbuf, g_hbm.at[pl.ds(off0 * HID, ccb * HID)])
                geom_chunk(off1, rid1, cid1, gbuf)
            b1.wait()
            w1 = pltpu.async_copy(buf1, o_hbm.at[pl.ds(off1, ccb)], sw1)
            if with_geom:
                pltpu.sync_copy(gbuf, g_hbm.at[pl.ds(off1 * HID, ccb * HID)])
            w0.wait()
            w1.wait()
            return carry

        lax.fori_loop(0, npair, pair, 0)
        if has_tail:
            do_chunk_sync(base + 2 * npair * ccb, rid0, cid0, buf0,
                          gbuf if with_geom else None, sa0, sb0, sw0)

    if with_ea:
        return k(A, B, row, col, *pxyz, ea)
    if with_geom:
        return k(A, B, row, col, *pxyz)
    return k(A, B, row, col)


def _segsum(vals, row, K, zeros):
    npr = N // 16

    @functools.partial(
        pl.kernel,
        out_type=jax.ShapeDtypeStruct((2 * N, K), jnp.float32),
        mesh=_sc_mesh(),
        scratch_types=[
            pltpu.VMEM((CCB,), jnp.int32),
            pltpu.VMEM((CCB,), jnp.int32),
            pltpu.VMEM((CCB, K), jnp.float32),
            pltpu.VMEM((CCB, K), jnp.float32),
            pltpu.VMEM_SHARED((N, K), jnp.float32),
        ] + [pltpu.SemaphoreType.DMA] * 4,
        compiler_params=pltpu.CompilerParams(use_tc_tiling_on_sc=False),
    )
    def k(vals_hbm, row_hbm, zeros_hbm, out_hbm,
          rid0, rid1, vbuf0, vbuf1, acc, sv0, sv1, ss0, ss1):
        ci = lax.axis_index("c")
        sid = lax.axis_index("s")
        wid = sid * 2 + ci
        pltpu.sync_copy(zeros_hbm.at[pl.ds(sid * npr, npr)],
                        acc.at[pl.ds(sid * npr, npr)])
        plsc.subcore_barrier()
        base = wid * EPW

        def pair(p, carry):
            off0 = base + p * 2 * CCB
            off1 = off0 + CCB
            pltpu.sync_copy(row_hbm.at[pl.ds(off0, CCB)], rid0)
            pltpu.sync_copy(row_hbm.at[pl.ds(off1, CCB)], rid1)
            v0 = pltpu.async_copy(vals_hbm.at[pl.ds(off0, CCB)], vbuf0, sv0)
            v1 = pltpu.async_copy(vals_hbm.at[pl.ds(off1, CCB)], vbuf1, sv1)
            v0.wait()
            c0 = pltpu.async_copy(vbuf0, acc.at[rid0], ss0, add=True)
            v1.wait()
            c1 = pltpu.async_copy(vbuf1, acc.at[rid1], ss1, add=True)
            c0.wait()
            c1.wait()
            return carry

        lax.fori_loop(0, NPAIR, pair, 0)
        off = base + 2 * NPAIR * CCB
        pltpu.sync_copy(row_hbm.at[pl.ds(off, CCB)], rid0)
        pltpu.sync_copy(vals_hbm.at[pl.ds(off, CCB)], vbuf0)
        pltpu.sync_copy(vbuf0, acc.at[rid0], add=True)
        plsc.subcore_barrier()
        pltpu.sync_copy(acc.at[pl.ds(sid * npr, npr)],
                        out_hbm.at[pl.ds(ci * N + sid * npr, npr)])

    return k(vals, row, zeros)


def _split_w1(W1):
    return W1[:HID], W1[HID:2 * HID], W1[2 * HID:]


def kernel(x, pos, mask, edge_attr, params, edge_index):
    row = edge_index[0]
    col = edge_index[1]
    blocks = params['blocks']

    gcl0 = blocks[0]['gcls'][0]
    W1_0 = gcl0['edge_mlp'][0][0]
    W1r0, W1c0, _ = _split_w1(W1_0)
    We, be = params['emb_in']
    h, A, B = _emb_in(x, We, be, W1r0, W1c0)
    zeros64 = jnp.zeros((N, HID), jnp.float32)
    ones_row = jnp.ones((1, HID), jnp.float32)
    SEL0 = jnp.zeros((HID, HID), jnp.float32).at[0, :].set(1.0)
    RSEL = _blockdiag(SEL0)
    F = None

    for bi, blk in enumerate(blocks):
        gcl = blk['gcls'][0]
        (W1, b1), (W2, b2) = gcl['edge_mlp']
        _, _, W1e = _split_w1(W1)
        pre, gflat = _combine(A, B, row, col,
                              (pos[:, 0], pos[:, 1], pos[:, 2]),
                              edge_attr if bi == 0 else None)
        G = gflat.reshape(E2, 128)
        if F is None:
            F = G
        W1e_pad = _pad_feat_w(W1e)
        m_p = _edge_mlp(pre.reshape(E2, 128), F,
                        _blockdiag(W1e_pad), jnp.concatenate([b1, b1]),
                        _blockdiag(W2), jnp.concatenate([b2, b2]))
        s2n = _segsum(m_p.reshape(E, HID), row, HID, zeros64)

        (Wn1, bn1), (Wn2, bn2) = gcl['node_mlp']
        Wn1h, Wn1a = Wn1[:HID], Wn1[HID:]
        (Wc1, bc1), (Wc2, bc2), (Wc3,) = blk['coord_mlp']
        Wc1r, Wc1c, Wc1e = _split_w1(Wc1)
        proj_ws = [Wc1r, Wc1c]
        if bi + 1 < len(blocks):
            gcl_n = blocks[bi + 1]['gcls'][0]
            W1n = gcl_n['edge_mlp'][0][0]
            W1rn, W1cn, _ = _split_w1(W1n)
            proj_ws += [W1rn, W1cn]
        outs = _node_update(h, s2n, mask, Wn1h, Wn1a, bn1, Wn2, bn2, proj_ws)
        h, Ac, Bc = outs[0], outs[1], outs[2]
        if bi + 1 < len(blocks):
            A, B = outs[3], outs[4]

        pre_c = _combine(Ac, Bc, row, col)
        Wc1e_pad = _pad_feat_w(Wc1e)
        W3sel = _blockdiag(Wc3 @ ones_row)
        trans_p = _coord_mlp(pre_c.reshape(E2, 128), F, G,
                             _blockdiag(Wc1e_pad), jnp.concatenate([bc1, bc1]),
                             _blockdiag(Wc2), jnp.concatenate([bc2, bc2]),
                             W3sel, RSEL)
        c2n = _segsum(trans_p.reshape(E, HID), row, HID, zeros64)
        cagg = c2n[:N, 1:4] + c2n[N:, 1:4]
        pos = pos + (cagg * (1.0 / NORM_FACTOR)) * mask
        h = h * mask

    Wo, bo = params['emb_out']
    return (_emb_out(h, Wo, bo, mask), pos)
